# baseline (speedup 1.0000x reference)
import functools

import jax
import jax.numpy as jnp
from jax import lax
from jax.experimental import pallas as pl
from jax.experimental.pallas import tpu as pltpu

D_MODEL = 1024
D_FF = 4096
TOKENS = 2 * 4096

_C = 8
_P = 10
_NB = 8
_INF_BITS = 0x7F800000


def _select_body(k_ref, we_ref, wd_ref, out_ref, state, cand, counts):
    p = pl.program_id(0)
    i = pl.program_id(1)
    k = k_ref[0]

    @pl.when(jnp.logical_and(p == 0, i == 0))
    def _init():
        state[0] = 0
        state[1] = _INF_BITS
        step = (_INF_BITS + _C) // (_C + 1)
        for j in range(_C):
            cand[j] = jnp.int32(min((j + 1) * step, _INF_BITS))
            counts[j] = 0

    be = lax.bitcast_convert_type(jnp.abs(we_ref[...]), jnp.int32)
    bd = lax.bitcast_convert_type(jnp.abs(wd_ref[...]), jnp.int32)
    for j in range(_C):
        c = cand[j]
        cnt = (jnp.sum((be >= c).astype(jnp.int32))
               + jnp.sum((bd >= c).astype(jnp.int32)))
        counts[j] = counts[j] + cnt

    @pl.when(i == _NB - 1)
    def _finalize():
        lo = state[0]
        hi = state[1]
        for j in range(_C):
            ge = counts[j] >= k
            lo = jnp.where(ge, jnp.maximum(lo, cand[j]), lo)
            hi = jnp.where(ge, hi, jnp.minimum(hi, cand[j]))
        state[0] = lo
        state[1] = hi
        step = (hi - lo + _C) // (_C + 1)
        for j in range(_C):
            cand[j] = jnp.minimum(lo + (j + 1) * step, hi)
            counts[j] = 0
        out_ref[...] = jnp.full((8, 128), lo, jnp.int32)


def _select(k_arr, W_enc, W_dec, interpret=False):
    grid_spec = pltpu.PrefetchScalarGridSpec(
        num_scalar_prefetch=1,
        grid=(_P, _NB),
        in_specs=[
            pl.BlockSpec((D_MODEL // _NB, D_FF), lambda p, i, k: (i, 0)),
            pl.BlockSpec((D_FF // _NB, D_MODEL), lambda p, i, k: (i, 0)),
        ],
        out_specs=pl.BlockSpec((8, 128), lambda p, i, k: (0, 0)),
        scratch_shapes=[
            pltpu.SMEM((2,), jnp.int32),
            pltpu.SMEM((_C,), jnp.int32),
            pltpu.SMEM((_C,), jnp.int32),
        ],
    )
    return pl.pallas_call(
        _select_body,
        grid_spec=grid_spec,
        out_shape=jax.ShapeDtypeStruct((8, 128), jnp.int32),
        interpret=interpret,
    )(k_arr, W_enc, W_dec)


_NC, _NS = 2, 16
_NW = _NC * _NS
_PER_W = (D_MODEL * D_FF) // _NW
_CHUNK = 16384
_NCH = _PER_W // _CHUNK
_NBINS = 4096
_HSIZE = 16 * _NBINS


def _sc_mesh():
    import jax.experimental.pallas.tpu_sc as plsc
    return plsc.VectorSubcoreMesh(core_axis_name="c", subcore_axis_name="s",
                                  num_cores=_NC, num_subcores=_NS)


def _sc_pass_body(p, *refs):
    from jax.experimental.pallas import tpu_sc as plsc
    if p == 0:
        (we, wd, k_hbm, hist_out, state_out,
         dbuf, hist, totals, rowbuf, sbuf, sem0, sem1) = refs
    else:
        (we, wd, histprev, stateprev, hist_out, state_out,
         dbuf, hist, totals, rowbuf, sbuf, sem0, sem1) = refs
    wid = lax.axis_index("s") * _NC + lax.axis_index("c")
    lane = lax.iota(jnp.int32, 16)
    ones = jnp.ones((16,), jnp.int32)
    zeros16 = jnp.zeros((16,), jnp.int32)

    if p == 0:
        pltpu.sync_copy(k_hbm, sbuf.at[0])
        kcur = sbuf[0, :]
        prefix = zeros16
    else:
        sems = (sem0, sem1)
        cp = pltpu.async_copy(histprev.at[0], rowbuf.at[0], sems[0])
        pltpu.sync_copy(histprev.at[1], totals)
        for r in range(_NW - 1):
            ncp = None
            if r + 1 < _NW - 1:
                ncp = pltpu.async_copy(histprev.at[r + 2],
                                       rowbuf.at[(r + 1) % 2],
                                       sems[(r + 1) % 2])
            cp.wait()
            rb = r % 2

            @plsc.parallel_loop(0, _NBINS // 16, unroll=4)
            def _addrow(j):
                totals[pl.ds(j * 16, 16)] = (totals[pl.ds(j * 16, 16)]
                                             + rowbuf[rb, pl.ds(j * 16, 16)])
            cp = ncp
        pltpu.sync_copy(stateprev, sbuf)
        pprev = sbuf[0, :]
        kprev = sbuf[1, :]

        def _find(jj, carry):
            sb, bv, kpv = carry
            j = (_NBINS // 16 - 1) - jj
            h = totals[pl.ds(j * 16, 16)]
            tot = jnp.full((16,), jnp.sum(h), jnp.int32)
            cs = plsc.cumsum(h)
            suf = tot - cs + h
            t = sb + suf
            m = jnp.logical_and(t >= kprev, (t - h) < kprev)
            bv = jnp.where(m, j * 16 + lane, bv)
            kpv = jnp.where(m, kprev - (t - h), kpv)
            return (sb + tot, bv, kpv)

        minus1 = jnp.full((16,), -1, jnp.int32)
        _, bv, kpv = lax.fori_loop(0, _NBINS // 16, _find,
                                   (zeros16, minus1, minus1))
        bcur = jnp.full((16,), jnp.max(bv), jnp.int32)
        kcur = jnp.full((16,), jnp.max(kpv), jnp.int32)
        prefix = pprev * _NBINS + bcur

    @plsc.parallel_loop(0, _HSIZE // 16, unroll=8)
    def _zero(i):
        hist[pl.ds(i * 16, 16)] = zeros16

    base = wid * _PER_W
    lane_base = lane * _NBINS

    def _srcref(c):
        if c < _NCH:
            return we.at[pl.ds(base + c * _CHUNK, _CHUNK)]
        return wd.at[pl.ds(base + (c - _NCH) * _CHUNK, _CHUNK)]

    sems = (sem0, sem1)
    cp = pltpu.async_copy(_srcref(0), dbuf.at[0], sems[0])
    for c in range(2 * _NCH):
        ncp = None
        if c + 1 < 2 * _NCH:
            ncp = pltpu.async_copy(_srcref(c + 1), dbuf.at[(c + 1) % 2],
                                   sems[(c + 1) % 2])
        cp.wait()
        cbuf = c % 2

        @plsc.parallel_loop(0, _CHUNK // 16, unroll=8)
        def _hstep(i):
            mag = dbuf[cbuf, pl.ds(i * 16, 16)] & jnp.int32(0x7FFFFFFF)
            if p == 0:
                bucket = lax.shift_right_logical(mag, 20)
                m = jnp.ones((16,), jnp.bool_)
            elif p == 1:
                bucket = lax.shift_right_logical(mag, 8) & jnp.int32(0xFFF)
                m = lax.shift_right_logical(mag, 20) == prefix
            else:
                bucket = mag & jnp.int32(0xFF)
                m = lax.shift_right_logical(mag, 8) == prefix
            plsc.addupdate_scatter(hist, [lane_base + bucket], ones, mask=m)
        cp = ncp

    @plsc.parallel_loop(0, _NBINS // 16, unroll=2)
    def _lred(j):
        a = hist[pl.ds(j * 16, 16)]
        for l in range(1, 16):
            a = a + hist[pl.ds(l * _NBINS + j * 16, 16)]
        totals[pl.ds(j * 16, 16)] = a
    pltpu.sync_copy(totals, hist_out.at[wid])

    @pl.when(wid == 0)
    def _wstate():
        sbuf[0, :] = prefix
        sbuf[1, :] = kcur
        pltpu.sync_copy(sbuf, state_out)


def _sc_scratch():
    return [
        pltpu.VMEM((2, _CHUNK), jnp.int32),
        pltpu.VMEM((_HSIZE,), jnp.int32),
        pltpu.VMEM((_NBINS,), jnp.int32),
        pltpu.VMEM((2, _NBINS), jnp.int32),
        pltpu.VMEM((2, 16), jnp.int32),
        pltpu.SemaphoreType.DMA,
        pltpu.SemaphoreType.DMA,
    ]


_SC_OUT = [jax.ShapeDtypeStruct((_NW, _NBINS), jnp.int32),
           jax.ShapeDtypeStruct((2, 16), jnp.int32)]


_SC_CPARAMS = pltpu.CompilerParams(needs_layout_passes=False)


def _sc_pass0(wef, wdf, k_vec):
    fn = pl.kernel(functools.partial(_sc_pass_body, 0), out_type=_SC_OUT,
                   mesh=_sc_mesh(), scratch_types=_sc_scratch(),
                   compiler_params=_SC_CPARAMS)
    return fn(wef, wdf, k_vec)


def _sc_pass12(p, wef, wdf, hprev, sprev):
    fn = pl.kernel(functools.partial(_sc_pass_body, p), out_type=_SC_OUT,
                   mesh=_sc_mesh(), scratch_types=_sc_scratch(),
                   compiler_params=_SC_CPARAMS)
    return fn(wef, wdf, hprev, sprev)


def _mask_body(t_ref, we_ref, wd_ref, weo_ref, wdo_ref):
    t = t_ref[0]
    we = we_ref[...]
    wd = wd_ref[...]
    weo_ref[...] = jnp.where(jnp.abs(we) >= t, we, 0.0).astype(jnp.bfloat16)
    wdo_ref[...] = jnp.where(jnp.abs(wd) >= t, wd, 0.0).astype(jnp.bfloat16)


def _mask(t_arr, W_enc, W_dec, interpret=False):
    nb = 8
    grid_spec = pltpu.PrefetchScalarGridSpec(
        num_scalar_prefetch=1,
        grid=(nb,),
        in_specs=[
            pl.BlockSpec((D_MODEL // nb, D_FF), lambda i, t: (i, 0)),
            pl.BlockSpec((D_FF // nb, D_MODEL), lambda i, t: (i, 0)),
        ],
        out_specs=[
            pl.BlockSpec((D_MODEL // nb, D_FF), lambda i, t: (i, 0)),
            pl.BlockSpec((D_FF // nb, D_MODEL), lambda i, t: (i, 0)),
        ],
    )
    return pl.pallas_call(
        _mask_body,
        grid_spec=grid_spec,
        out_shape=[
            jax.ShapeDtypeStruct((D_MODEL, D_FF), jnp.bfloat16),
            jax.ShapeDtypeStruct((D_FF, D_MODEL), jnp.bfloat16),
        ],
        interpret=interpret,
    )(t_arr, W_enc, W_dec)


def _maskfin_body(h3_ref, st_ref, we_ref, wd_ref, weo_ref, wdo_ref):
    hs = jnp.sum(h3_ref[...], axis=0).astype(jnp.float32)
    rows = jnp.dot(
        (lax.broadcasted_iota(jnp.int32, (32, 32), 1)
         < lax.broadcasted_iota(jnp.int32, (32, 32), 0)).astype(jnp.float32),
        hs, preferred_element_type=jnp.float32)
    rowoff = jnp.sum(rows, axis=1, keepdims=True)
    upper = (lax.broadcasted_iota(jnp.int32, (128, 128), 0)
             < lax.broadcasted_iota(jnp.int32, (128, 128), 1)
             ).astype(jnp.float32)
    within = jnp.dot(hs, upper, preferred_element_type=jnp.float32)
    pe = rowoff + within
    total = jnp.sum(hs)
    target = total - st_ref[1, 0].astype(jnp.float32)
    cnt = jnp.sum((pe <= target).astype(jnp.float32))
    bfin = cnt.astype(jnp.int32) - 1
    tbits = st_ref[0, 0] * 256 + bfin
    t = lax.bitcast_convert_type(tbits, jnp.float32)
    we = we_ref[...]
    wd = wd_ref[...]
    weo_ref[...] = jnp.where(jnp.abs(we) >= t, we, 0.0).astype(jnp.bfloat16)
    wdo_ref[...] = jnp.where(jnp.abs(wd) >= t, wd, 0.0).astype(jnp.bfloat16)


def _maskfin(h3, state, W_enc, W_dec, interpret=False):
    nb = 8
    return pl.pallas_call(
        _maskfin_body,
        grid=(nb,),
        in_specs=[
            pl.BlockSpec((_NW, 32, 128), lambda i: (0, 0, 0)),
            pl.BlockSpec((2, 16), lambda i: (0, 0)),
            pl.BlockSpec((D_MODEL // nb, D_FF), lambda i: (i, 0)),
            pl.BlockSpec((D_FF // nb, D_MODEL), lambda i: (i, 0)),
        ],
        out_specs=[
            pl.BlockSpec((D_MODEL // nb, D_FF), lambda i: (i, 0)),
            pl.BlockSpec((D_FF // nb, D_MODEL), lambda i: (i, 0)),
        ],
        out_shape=[
            jax.ShapeDtypeStruct((D_MODEL, D_FF), jnp.bfloat16),
            jax.ShapeDtypeStruct((D_FF, D_MODEL), jnp.bfloat16),
        ],
        interpret=interpret,
    )(h3, state, W_enc, W_dec)


def _xcast_body(x_ref, xo_ref):
    xo_ref[...] = x_ref[...].astype(jnp.bfloat16)


def _xcast(xf, interpret=False):
    nb = 4
    return pl.pallas_call(
        _xcast_body,
        grid=(nb,),
        in_specs=[pl.BlockSpec((TOKENS // nb, D_MODEL), lambda i: (i, 0))],
        out_specs=pl.BlockSpec((TOKENS // nb, D_MODEL), lambda i: (i, 0)),
        out_shape=jax.ShapeDtypeStruct((TOKENS, D_MODEL), jnp.bfloat16),
        interpret=interpret,
    )(xf)


_BT = 2048
_BF = 512


def _mlp_body(x_ref, we_ref, wd_ref, y_ref):
    f = pl.program_id(1)
    h = jnp.dot(x_ref[...], we_ref[...], preferred_element_type=jnp.float32)
    h = jnp.maximum(h, 0.0).astype(jnp.bfloat16)
    yb = jnp.dot(h, wd_ref[...], preferred_element_type=jnp.float32)

    @pl.when(f == 0)
    def _first():
        y_ref[...] = yb

    @pl.when(f > 0)
    def _acc():
        y_ref[...] += yb


def _mlp(xf, We_b, Wd_b, interpret=False):
    grid = (TOKENS // _BT, D_FF // _BF)
    return pl.pallas_call(
        _mlp_body,
        grid=grid,
        in_specs=[
            pl.BlockSpec((_BT, D_MODEL), lambda t, f: (t, 0)),
            pl.BlockSpec((D_MODEL, _BF), lambda t, f: (0, f)),
            pl.BlockSpec((_BF, D_MODEL), lambda t, f: (f, 0)),
        ],
        out_specs=pl.BlockSpec((_BT, D_MODEL), lambda t, f: (t, 0)),
        out_shape=jax.ShapeDtypeStruct((TOKENS, D_MODEL), jnp.float32),
        interpret=interpret,
    )(xf, We_b, Wd_b)


def kernel(x, W_enc, W_dec, k):
    k_vec = jnp.full((16,), k, jnp.int32)
    wef = lax.bitcast_convert_type(W_enc, jnp.int32).reshape(-1)
    wdf = lax.bitcast_convert_type(W_dec, jnp.int32).reshape(-1)
    xb = _xcast(x.reshape(TOKENS, D_MODEL))
    h0, s0 = _sc_pass0(wef, wdf, k_vec)
    h1, s1 = _sc_pass12(1, wef, wdf, h0, s0)
    h2, s2 = _sc_pass12(2, wef, wdf, h1, s1)
    We_b, Wd_b = _maskfin(h2.reshape(_NW, 32, 128), s2, W_enc, W_dec)
    y = _mlp(xb, We_b, Wd_b)
    return y.reshape(x.shape)


# speedup vs baseline: 1.1079x; 1.1079x over previous
import functools

import jax
import jax.numpy as jnp
from jax import lax
from jax.experimental import pallas as pl
from jax.experimental.pallas import tpu as pltpu

D_MODEL = 1024
D_FF = 4096
TOKENS = 2 * 4096

_C = 8
_P = 10
_NB = 8
_INF_BITS = 0x7F800000


def _select_body(k_ref, we_ref, wd_ref, out_ref, state, cand, counts):
    p = pl.program_id(0)
    i = pl.program_id(1)
    k = k_ref[0]

    @pl.when(jnp.logical_and(p == 0, i == 0))
    def _init():
        state[0] = 0
        state[1] = _INF_BITS
        step = (_INF_BITS + _C) // (_C + 1)
        for j in range(_C):
            cand[j] = jnp.int32(min((j + 1) * step, _INF_BITS))
            counts[j] = 0

    be = lax.bitcast_convert_type(jnp.abs(we_ref[...]), jnp.int32)
    bd = lax.bitcast_convert_type(jnp.abs(wd_ref[...]), jnp.int32)
    for j in range(_C):
        c = cand[j]
        cnt = (jnp.sum((be >= c).astype(jnp.int32))
               + jnp.sum((bd >= c).astype(jnp.int32)))
        counts[j] = counts[j] + cnt

    @pl.when(i == _NB - 1)
    def _finalize():
        lo = state[0]
        hi = state[1]
        for j in range(_C):
            ge = counts[j] >= k
            lo = jnp.where(ge, jnp.maximum(lo, cand[j]), lo)
            hi = jnp.where(ge, hi, jnp.minimum(hi, cand[j]))
        state[0] = lo
        state[1] = hi
        step = (hi - lo + _C) // (_C + 1)
        for j in range(_C):
            cand[j] = jnp.minimum(lo + (j + 1) * step, hi)
            counts[j] = 0
        out_ref[...] = jnp.full((8, 128), lo, jnp.int32)


def _select(k_arr, W_enc, W_dec, interpret=False):
    grid_spec = pltpu.PrefetchScalarGridSpec(
        num_scalar_prefetch=1,
        grid=(_P, _NB),
        in_specs=[
            pl.BlockSpec((D_MODEL // _NB, D_FF), lambda p, i, k: (i, 0)),
            pl.BlockSpec((D_FF // _NB, D_MODEL), lambda p, i, k: (i, 0)),
        ],
        out_specs=pl.BlockSpec((8, 128), lambda p, i, k: (0, 0)),
        scratch_shapes=[
            pltpu.SMEM((2,), jnp.int32),
            pltpu.SMEM((_C,), jnp.int32),
            pltpu.SMEM((_C,), jnp.int32),
        ],
    )
    return pl.pallas_call(
        _select_body,
        grid_spec=grid_spec,
        out_shape=jax.ShapeDtypeStruct((8, 128), jnp.int32),
        interpret=interpret,
    )(k_arr, W_enc, W_dec)


_NC, _NS = 2, 16
_NW = _NC * _NS
_NBINS = 2048
_HSIZE = 16 * _NBINS
_ER = D_MODEL // _NW
_DR = D_FF // _NW
_ECH, _ERB = 4, 8
_DCH, _DRB = 16, 8


def _sc_mesh():
    import jax.experimental.pallas.tpu_sc as plsc
    return plsc.VectorSubcoreMesh(core_axis_name="c", subcore_axis_name="s",
                                  num_cores=_NC, num_subcores=_NS)


def _sc_pass_body(p, *refs):
    from jax.experimental.pallas import tpu_sc as plsc
    if p == 0:
        (we, wd, k_hbm, hist_out, state_out,
         dbe, dbd, hist, totals, rowbuf, sbuf, sem0, sem1) = refs
    else:
        (we, wd, histprev, stateprev, hist_out, state_out,
         dbe, dbd, hist, totals, rowbuf, sbuf, sem0, sem1) = refs
    wid = lax.axis_index("s") * _NC + lax.axis_index("c")
    lane = lax.iota(jnp.int32, 16)
    ones = jnp.ones((16,), jnp.int32)
    zeros16 = jnp.zeros((16,), jnp.int32)
    sems = (sem0, sem1)

    if p == 0:
        pltpu.sync_copy(k_hbm, sbuf.at[0])
        kcur = sbuf[0, :]
        prefix = zeros16
    else:
        cp = pltpu.async_copy(histprev.at[0], rowbuf.at[0], sems[0])
        pltpu.sync_copy(histprev.at[1], totals)
        for r in range(_NW - 1):
            ncp = None
            if r + 1 < _NW - 1:
                ncp = pltpu.async_copy(histprev.at[r + 2],
                                       rowbuf.at[(r + 1) % 2],
                                       sems[(r + 1) % 2])
            cp.wait()
            rb = r % 2

            @plsc.parallel_loop(0, _NBINS // 16, unroll=4)
            def _addrow(j):
                totals[pl.ds(j * 16, 16)] = (totals[pl.ds(j * 16, 16)]
                                             + rowbuf[rb, pl.ds(j * 16, 16)])
            cp = ncp
        pltpu.sync_copy(stateprev, sbuf)
        pprev = sbuf[0, :]
        kprev = sbuf[1, :]

        def _find(jj, carry):
            sb, bv, kpv = carry
            j = (_NBINS // 16 - 1) - jj
            h = totals[pl.ds(j * 16, 16)]
            tot = jnp.full((16,), jnp.sum(h), jnp.int32)
            cs = plsc.cumsum(h)
            suf = tot - cs + h
            t = sb + suf
            m = jnp.logical_and(t >= kprev, (t - h) < kprev)
            bv = jnp.where(m, j * 16 + lane, bv)
            kpv = jnp.where(m, kprev - (t - h), kpv)
            return (sb + tot, bv, kpv)

        minus1 = jnp.full((16,), -1, jnp.int32)
        _, bv, kpv = lax.fori_loop(0, _NBINS // 16, _find,
                                   (zeros16, minus1, minus1))
        bcur = jnp.full((16,), jnp.max(bv), jnp.int32)
        kcur = jnp.full((16,), jnp.max(kpv), jnp.int32)
        prefix = pprev * _NBINS + bcur

    @plsc.parallel_loop(0, _HSIZE // 16, unroll=8)
    def _zero(i):
        hist[pl.ds(i * 16, 16)] = zeros16

    lane_base = lane * _NBINS

    def _scat(mag):
        if p == 0:
            bucket = lax.shift_right_logical(mag, 21)
            m = jnp.ones((16,), jnp.bool_)
        elif p == 1:
            bucket = lax.shift_right_logical(mag, 10) & jnp.int32(0x7FF)
            m = lax.shift_right_logical(mag, 21) == prefix
        else:
            bucket = mag & jnp.int32(0x3FF)
            m = lax.shift_right_logical(mag, 10) == prefix
        plsc.addupdate_scatter(hist, [lane_base + bucket], ones, mask=m)

    for mat, db, nch, nrb, ncols, rbase in (
            (we, dbe, _ECH, _ERB, D_FF, wid * _ER),
            (wd, dbd, _DCH, _DRB, D_MODEL, wid * _DR)):
        cp = pltpu.async_copy(mat.at[pl.ds(rbase, nrb), :], db.at[0], sems[0])
        for c in range(nch):
            ncp = None
            if c + 1 < nch:
                ncp = pltpu.async_copy(
                    mat.at[pl.ds(rbase + (c + 1) * nrb, nrb), :],
                    db.at[(c + 1) % 2], sems[(c + 1) % 2])
            cp.wait()
            cbuf = c % 2

            @plsc.parallel_loop(0, ncols // 16, unroll=2)
            def _hstep(i):
                for r in range(nrb):
                    mag = db[cbuf, r, pl.ds(i * 16, 16)] & jnp.int32(0x7FFFFFFF)
                    _scat(mag)
            cp = ncp

    @plsc.parallel_loop(0, _NBINS // 16, unroll=2)
    def _lred(j):
        a = hist[pl.ds(j * 16, 16)]
        for l in range(1, 16):
            a = a + hist[pl.ds(l * _NBINS + j * 16, 16)]
        totals[pl.ds(j * 16, 16)] = a
    pltpu.sync_copy(totals, hist_out.at[wid])

    @pl.when(wid == 0)
    def _wstate():
        sbuf[0, :] = prefix
        sbuf[1, :] = kcur
        pltpu.sync_copy(sbuf, state_out)


def _sc_scratch():
    return [
        pltpu.VMEM((2, _ERB, D_FF), jnp.int32),
        pltpu.VMEM((2, _DRB, D_MODEL), jnp.int32),
        pltpu.VMEM((_HSIZE,), jnp.int32),
        pltpu.VMEM((_NBINS,), jnp.int32),
        pltpu.VMEM((2, _NBINS), jnp.int32),
        pltpu.VMEM((2, 16), jnp.int32),
        pltpu.SemaphoreType.DMA,
        pltpu.SemaphoreType.DMA,
    ]


_SC_OUT = [jax.ShapeDtypeStruct((_NW, _NBINS), jnp.int32),
           jax.ShapeDtypeStruct((2, 16), jnp.int32)]


_SC_CPARAMS = pltpu.CompilerParams(needs_layout_passes=False)


def _sc_pass0(wef, wdf, k_vec):
    fn = pl.kernel(functools.partial(_sc_pass_body, 0), out_type=_SC_OUT,
                   mesh=_sc_mesh(), scratch_types=_sc_scratch(),
                   compiler_params=_SC_CPARAMS)
    return fn(wef, wdf, k_vec)


def _sc_pass12(p, wef, wdf, hprev, sprev):
    fn = pl.kernel(functools.partial(_sc_pass_body, p), out_type=_SC_OUT,
                   mesh=_sc_mesh(), scratch_types=_sc_scratch(),
                   compiler_params=_SC_CPARAMS)
    return fn(wef, wdf, hprev, sprev)


def _maskfin_body(h3_ref, st_ref, we_ref, wd_ref, weo_ref, wdo_ref):
    hs = jnp.sum(h3_ref[...], axis=0).astype(jnp.float32)
    rows = jnp.dot(
        (lax.broadcasted_iota(jnp.int32, (16, 16), 1)
         < lax.broadcasted_iota(jnp.int32, (16, 16), 0)).astype(jnp.float32),
        hs, preferred_element_type=jnp.float32)
    rowoff = jnp.sum(rows, axis=1, keepdims=True)
    upper = (lax.broadcasted_iota(jnp.int32, (128, 128), 0)
             < lax.broadcasted_iota(jnp.int32, (128, 128), 1)
             ).astype(jnp.float32)
    within = jnp.dot(hs, upper, preferred_element_type=jnp.float32)
    pe = rowoff + within
    total = jnp.sum(hs)
    target = total - st_ref[1, 0].astype(jnp.float32)
    cnt = jnp.sum((pe <= target).astype(jnp.float32))
    bfin = cnt.astype(jnp.int32) - 1
    tbits = st_ref[0, 0] * 1024 + bfin
    t = lax.bitcast_convert_type(tbits, jnp.float32)
    we = we_ref[...]
    wd = wd_ref[...]
    weo_ref[...] = jnp.where(jnp.abs(we) >= t, we, 0.0).astype(jnp.bfloat16)
    wdo_ref[...] = jnp.where(jnp.abs(wd) >= t, wd, 0.0).astype(jnp.bfloat16)


def _maskfin(h3, state, W_enc, W_dec, interpret=False):
    nb = 8
    return pl.pallas_call(
        _maskfin_body,
        grid=(nb,),
        in_specs=[
            pl.BlockSpec((_NW, 16, 128), lambda i: (0, 0, 0)),
            pl.BlockSpec((2, 16), lambda i: (0, 0)),
            pl.BlockSpec((D_MODEL // nb, D_FF), lambda i: (i, 0)),
            pl.BlockSpec((D_FF // nb, D_MODEL), lambda i: (i, 0)),
        ],
        out_specs=[
            pl.BlockSpec((D_MODEL // nb, D_FF), lambda i: (i, 0)),
            pl.BlockSpec((D_FF // nb, D_MODEL), lambda i: (i, 0)),
        ],
        out_shape=[
            jax.ShapeDtypeStruct((D_MODEL, D_FF), jnp.bfloat16),
            jax.ShapeDtypeStruct((D_FF, D_MODEL), jnp.bfloat16),
        ],
        interpret=interpret,
    )(h3, state, W_enc, W_dec)


def _xcast_body(x_ref, xo_ref):
    xo_ref[...] = x_ref[...].astype(jnp.bfloat16)


def _xcast(xf, interpret=False):
    nb = 4
    return pl.pallas_call(
        _xcast_body,
        grid=(nb,),
        in_specs=[pl.BlockSpec((TOKENS // nb, D_MODEL), lambda i: (i, 0))],
        out_specs=pl.BlockSpec((TOKENS // nb, D_MODEL), lambda i: (i, 0)),
        out_shape=jax.ShapeDtypeStruct((TOKENS, D_MODEL), jnp.bfloat16),
        interpret=interpret,
    )(xf)


_BT = 2048
_BF = 512


def _mlp_body(x_ref, we_ref, wd_ref, y_ref):
    f = pl.program_id(1)
    xb = x_ref[...].astype(jnp.bfloat16)
    h = jnp.dot(xb, we_ref[...], preferred_element_type=jnp.float32)
    h = jnp.maximum(h, 0.0).astype(jnp.bfloat16)
    yb = jnp.dot(h, wd_ref[...], preferred_element_type=jnp.float32)

    @pl.when(f == 0)
    def _first():
        y_ref[...] = yb

    @pl.when(f > 0)
    def _acc():
        y_ref[...] += yb


def _mlp(xf, We_b, Wd_b, interpret=False):
    grid = (TOKENS // _BT, D_FF // _BF)
    return pl.pallas_call(
        _mlp_body,
        grid=grid,
        in_specs=[
            pl.BlockSpec((_BT, D_MODEL), lambda t, f: (t, 0)),
            pl.BlockSpec((D_MODEL, _BF), lambda t, f: (0, f)),
            pl.BlockSpec((_BF, D_MODEL), lambda t, f: (f, 0)),
        ],
        out_specs=pl.BlockSpec((_BT, D_MODEL), lambda t, f: (t, 0)),
        out_shape=jax.ShapeDtypeStruct((TOKENS, D_MODEL), jnp.float32),
        interpret=interpret,
    )(xf, We_b, Wd_b)


def kernel(x, W_enc, W_dec, k):
    k_vec = jnp.full((16,), k, jnp.int32)
    wef = lax.bitcast_convert_type(W_enc, jnp.int32)
    wdf = lax.bitcast_convert_type(W_dec, jnp.int32)
    h0, s0 = _sc_pass0(wef, wdf, k_vec)
    h1, s1 = _sc_pass12(1, wef, wdf, h0, s0)
    h2, s2 = _sc_pass12(2, wef, wdf, h1, s1)
    We_b, Wd_b = _maskfin(h2.reshape(_NW, 16, 128), s2, W_enc, W_dec)
    y = _mlp(x.reshape(TOKENS, D_MODEL), We_b, Wd_b)
    return y.reshape(x.shape)
